# baseline (device time: 24924 ns/iter reference)
import jax
import jax.numpy as jnp
from jax import lax
from jax.experimental import pallas as pl
from jax.experimental.pallas import tpu as pltpu

TM = 256


def kernel(x, dy, gamma):
    del gamma
    m, d = x.shape
    n_tiles = m // TM

    def body(x_ref, dy_ref, out_ref, accum, recvs, ssems, rsems):
        i = pl.program_id(0)
        my_x = lax.axis_index("x")
        my_y = lax.axis_index("y")
        my_z = lax.axis_index("z")

        @pl.when(i == 0)
        def _():
            accum[...] = jnp.zeros_like(accum)
            barrier = pltpu.get_barrier_semaphore()
            for k in (1, 2, 3):
                pl.semaphore_signal(
                    barrier,
                    inc=1,
                    device_id=(my_x, my_y, (my_z + k) % 4),
                    device_id_type=pl.DeviceIdType.MESH,
                )
            pl.semaphore_wait(barrier, 3)

        xt = x_ref[...]
        dyt = dy_ref[...]
        x2 = xt * xt
        dyx = dyt * xt
        ones_col = jnp.ones((d, 1), jnp.float32)
        s1 = jnp.dot(xt, ones_col, preferred_element_type=jnp.float32)
        s2 = jnp.dot(x2, ones_col, preferred_element_type=jnp.float32)
        mu = s1 * (1.0 / d)
        var = s2 * (1.0 / d) - mu * mu
        rstd = lax.rsqrt(var + 1e-5)
        b = mu * rstd
        contract_rows = (((0,), (0,)), ((), ()))
        p1 = lax.dot_general(
            rstd, dyx, contract_rows, preferred_element_type=jnp.float32
        )
        w2 = jnp.concatenate([b, jnp.ones_like(b)], axis=1)
        p2 = lax.dot_general(
            w2, dyt, contract_rows, preferred_element_type=jnp.float32
        )
        dg = p1 - p2[0:1]
        accum[...] += jnp.concatenate([dg, p2[1:2]], axis=0)

        @pl.when(i == n_tiles - 1)
        def _():
            rdmas = []
            for k in (1, 2, 3):
                rdma = pltpu.make_async_remote_copy(
                    src_ref=accum,
                    dst_ref=recvs.at[k - 1],
                    send_sem=ssems.at[k - 1],
                    recv_sem=rsems.at[k - 1],
                    device_id=(my_x, my_y, (my_z + k) % 4),
                    device_id_type=pl.DeviceIdType.MESH,
                )
                rdma.start()
                rdmas.append(rdma)
            total = accum[...]
            for k, rdma in zip((1, 2, 3), rdmas):
                rdma.wait_send()
                rdma.wait_recv()
                total = total + recvs[k - 1]
            out_ref[...] = total

    return pl.pallas_call(
        body,
        grid=(n_tiles,),
        in_specs=[
            pl.BlockSpec((TM, d), lambda i: (i, 0)),
            pl.BlockSpec((TM, d), lambda i: (i, 0)),
        ],
        out_specs=pl.BlockSpec((2, d), lambda i: (0, 0)),
        out_shape=jax.ShapeDtypeStruct((2, d), jnp.float32),
        scratch_shapes=[
            pltpu.VMEM((2, d), jnp.float32),
            pltpu.VMEM((3, 2, d), jnp.float32),
            pltpu.SemaphoreType.DMA((3,)),
            pltpu.SemaphoreType.DMA((3,)),
        ],
        compiler_params=pltpu.CompilerParams(collective_id=0),
    )(x, dy)


# device time: 19559 ns/iter; 1.2743x vs baseline; 1.2743x over previous
import jax
import jax.numpy as jnp
from jax import lax
from jax.experimental import pallas as pl
from jax.experimental.pallas import tpu as pltpu

TM = 512
XS = 2
ZS = 4

_OFFSETS = tuple(
    (dx, dz) for dx in range(XS) for dz in range(ZS) if (dx, dz) != (0, 0)
)


def kernel(x, dy, gamma):
    del gamma
    m, d = x.shape
    m_loc = m // XS
    n_tiles = m_loc // TM

    def body(off_ref, x_ref, dy_ref, out_ref, accum, recvs, ssems, rsems):
        i = pl.program_id(0)
        my_x = lax.axis_index("x")
        my_y = lax.axis_index("y")
        my_z = lax.axis_index("z")

        @pl.when(i == 0)
        def _():
            accum[...] = jnp.zeros_like(accum)

        xt = x_ref[...]
        dyt = dy_ref[...]
        mu = jnp.mean(xt, axis=1, keepdims=True)
        xc = xt - mu
        var = jnp.mean(xc * xc, axis=1, keepdims=True)
        rstd = lax.rsqrt(var + 1e-5)
        xhat = xc * rstd
        dg = jnp.sum(dyt * xhat, axis=0, keepdims=True)
        db = jnp.sum(dyt, axis=0, keepdims=True)
        accum[...] += jnp.concatenate([dg, db], axis=0)

        @pl.when(i == 0)
        def _():
            barrier = pltpu.get_barrier_semaphore()
            for dx, dz in _OFFSETS:
                pl.semaphore_signal(
                    barrier,
                    inc=1,
                    device_id=((my_x + dx) % XS, my_y, (my_z + dz) % ZS),
                    device_id_type=pl.DeviceIdType.MESH,
                )
            pl.semaphore_wait(barrier, len(_OFFSETS))

        @pl.when(i == n_tiles - 1)
        def _():
            rdmas = []
            for s, (dx, dz) in enumerate(_OFFSETS):
                rdma = pltpu.make_async_remote_copy(
                    src_ref=accum,
                    dst_ref=recvs.at[s],
                    send_sem=ssems.at[s],
                    recv_sem=rsems.at[s],
                    device_id=((my_x + dx) % XS, my_y, (my_z + dz) % ZS),
                    device_id_type=pl.DeviceIdType.MESH,
                )
                rdma.start()
                rdmas.append(rdma)
            total = accum[...]
            for s, rdma in enumerate(rdmas):
                rdma.wait_send()
                rdma.wait_recv()
                total = total + recvs[s]
            out_ref[...] = total

    n_peers = len(_OFFSETS)
    grid_spec = pltpu.PrefetchScalarGridSpec(
        num_scalar_prefetch=1,
        grid=(n_tiles,),
        in_specs=[
            pl.BlockSpec((TM, d), lambda i, off: (off[0] + i, 0)),
            pl.BlockSpec((TM, d), lambda i, off: (off[0] + i, 0)),
        ],
        out_specs=pl.BlockSpec((2, d), lambda i, off: (0, 0)),
        scratch_shapes=[
            pltpu.VMEM((2, d), jnp.float32),
            pltpu.VMEM((n_peers, 2, d), jnp.float32),
            pltpu.SemaphoreType.DMA((n_peers,)),
            pltpu.SemaphoreType.DMA((n_peers,)),
        ],
    )
    off = jnp.reshape(lax.axis_index("x") * n_tiles, (1,)).astype(jnp.int32)
    return pl.pallas_call(
        body,
        grid_spec=grid_spec,
        out_shape=jax.ShapeDtypeStruct((2, d), jnp.float32),
        compiler_params=pltpu.CompilerParams(collective_id=0),
    )(off, x, dy)


# device time: 18836 ns/iter; 1.3232x vs baseline; 1.0384x over previous
import jax
import jax.numpy as jnp
from jax import lax
from jax.experimental import pallas as pl
from jax.experimental.pallas import tpu as pltpu

TM = 512
XS = 2
ZS = 4

_OFFSETS = tuple(
    (dx, dz) for dx in range(XS) for dz in range(ZS) if (dx, dz) != (0, 0)
)


def kernel(x, dy, gamma):
    del gamma
    m, d = x.shape
    m_loc = m // XS
    n_tiles = m_loc // TM

    def body(off_ref, x_ref, dy_ref, out_ref, accum, recvs, ssems, rsems):
        i = pl.program_id(0)
        my_x = lax.axis_index("x")
        my_y = lax.axis_index("y")
        my_z = lax.axis_index("z")

        @pl.when(i == 0)
        def _():
            accum[...] = jnp.zeros_like(accum)
            barrier = pltpu.get_barrier_semaphore()
            for dx, dz in _OFFSETS:
                pl.semaphore_signal(
                    barrier,
                    inc=1,
                    device_id=((my_x + dx) % XS, my_y, (my_z + dz) % ZS),
                    device_id_type=pl.DeviceIdType.MESH,
                )

        xt = x_ref[...]
        dyt = dy_ref[...]
        mu = jnp.mean(xt, axis=1, keepdims=True)
        xc = xt - mu
        var = jnp.mean(xc * xc, axis=1, keepdims=True)
        rstd = lax.rsqrt(var + 1e-5)
        xhat = xc * rstd
        dg = jnp.sum(dyt * xhat, axis=0, keepdims=True)
        db = jnp.sum(dyt, axis=0, keepdims=True)
        accum[...] += jnp.concatenate([dg, db], axis=0)

        @pl.when(i == n_tiles - 1)
        def _():
            barrier = pltpu.get_barrier_semaphore()
            pl.semaphore_wait(barrier, len(_OFFSETS))
            rdmas = []
            for s, (dx, dz) in enumerate(_OFFSETS):
                rdma = pltpu.make_async_remote_copy(
                    src_ref=accum,
                    dst_ref=recvs.at[s],
                    send_sem=ssems.at[s],
                    recv_sem=rsems.at[s],
                    device_id=((my_x + dx) % XS, my_y, (my_z + dz) % ZS),
                    device_id_type=pl.DeviceIdType.MESH,
                )
                rdma.start()
                rdmas.append(rdma)
            total = accum[...]
            for s, rdma in enumerate(rdmas):
                rdma.wait_send()
                rdma.wait_recv()
                total = total + recvs[s]
            out_ref[...] = total

    n_peers = len(_OFFSETS)
    grid_spec = pltpu.PrefetchScalarGridSpec(
        num_scalar_prefetch=1,
        grid=(n_tiles,),
        in_specs=[
            pl.BlockSpec((TM, d), lambda i, off: (off[0] + i, 0)),
            pl.BlockSpec((TM, d), lambda i, off: (off[0] + i, 0)),
        ],
        out_specs=pl.BlockSpec((2, d), lambda i, off: (0, 0)),
        scratch_shapes=[
            pltpu.VMEM((2, d), jnp.float32),
            pltpu.VMEM((n_peers, 2, d), jnp.float32),
            pltpu.SemaphoreType.DMA((n_peers,)),
            pltpu.SemaphoreType.DMA((n_peers,)),
        ],
    )
    off = jnp.reshape(lax.axis_index("x") * n_tiles, (1,)).astype(jnp.int32)
    return pl.pallas_call(
        body,
        grid_spec=grid_spec,
        out_shape=jax.ShapeDtypeStruct((2, d), jnp.float32),
        compiler_params=pltpu.CompilerParams(collective_id=0),
    )(off, x, dy)


# device time: 18404 ns/iter; 1.3543x vs baseline; 1.0235x over previous
import jax
import jax.numpy as jnp
from jax import lax
from jax.experimental import pallas as pl
from jax.experimental.pallas import tpu as pltpu

TM = 256
XS = 2
ZS = 4

_OFFSETS = tuple(
    (dx, dz) for dx in range(XS) for dz in range(ZS) if (dx, dz) != (0, 0)
)


def kernel(x, dy, gamma):
    del gamma
    m, d = x.shape
    m_loc = m // XS
    n_tiles = m_loc // TM
    n_peers = len(_OFFSETS)

    def body(x_hbm, dy_hbm, out_ref, xb, dyb, accum, recvs, ssems, rsems, csems):
        my_x = lax.axis_index("x")
        my_y = lax.axis_index("y")
        my_z = lax.axis_index("z")

        barrier = pltpu.get_barrier_semaphore()
        for dx, dz in _OFFSETS:
            pl.semaphore_signal(
                barrier,
                inc=1,
                device_id=((my_x + dx) % XS, my_y, (my_z + dz) % ZS),
                device_id_type=pl.DeviceIdType.MESH,
            )

        row0 = my_x * m_loc
        copies = []
        for t in range(n_tiles):
            cx = pltpu.make_async_copy(
                x_hbm.at[pl.ds(row0 + t * TM, TM), :], xb.at[t], csems.at[2 * t]
            )
            cd = pltpu.make_async_copy(
                dy_hbm.at[pl.ds(row0 + t * TM, TM), :],
                dyb.at[t],
                csems.at[2 * t + 1],
            )
            cx.start()
            cd.start()
            copies.append((cx, cd))

        total = jnp.zeros((2, d), jnp.float32)
        for t in range(n_tiles):
            cx, cd = copies[t]
            cx.wait()
            cd.wait()
            xt = xb[t]
            dyt = dyb[t]
            mu = jnp.mean(xt, axis=1, keepdims=True)
            xc = xt - mu
            var = jnp.mean(xc * xc, axis=1, keepdims=True)
            rstd = lax.rsqrt(var + 1e-5)
            xhat = xc * rstd
            dg = jnp.sum(dyt * xhat, axis=0, keepdims=True)
            db = jnp.sum(dyt, axis=0, keepdims=True)
            total = total + jnp.concatenate([dg, db], axis=0)
        accum[...] = total

        pl.semaphore_wait(barrier, n_peers)
        rdmas = []
        for s, (dx, dz) in enumerate(_OFFSETS):
            rdma = pltpu.make_async_remote_copy(
                src_ref=accum,
                dst_ref=recvs.at[s],
                send_sem=ssems.at[s],
                recv_sem=rsems.at[s],
                device_id=((my_x + dx) % XS, my_y, (my_z + dz) % ZS),
                device_id_type=pl.DeviceIdType.MESH,
            )
            rdma.start()
            rdmas.append(rdma)
        for s, rdma in enumerate(rdmas):
            rdma.wait_send()
            rdma.wait_recv()
            total = total + recvs[s]
        out_ref[...] = total

    return pl.pallas_call(
        body,
        in_specs=[
            pl.BlockSpec(memory_space=pltpu.MemorySpace.HBM),
            pl.BlockSpec(memory_space=pltpu.MemorySpace.HBM),
        ],
        out_specs=pl.BlockSpec(memory_space=pltpu.MemorySpace.VMEM),
        out_shape=jax.ShapeDtypeStruct((2, d), jnp.float32),
        scratch_shapes=[
            pltpu.VMEM((n_tiles, TM, d), jnp.float32),
            pltpu.VMEM((n_tiles, TM, d), jnp.float32),
            pltpu.VMEM((2, d), jnp.float32),
            pltpu.VMEM((n_peers, 2, d), jnp.float32),
            pltpu.SemaphoreType.DMA((n_peers,)),
            pltpu.SemaphoreType.DMA((n_peers,)),
            pltpu.SemaphoreType.DMA((2 * n_tiles,)),
        ],
        compiler_params=pltpu.CompilerParams(collective_id=0),
    )(x, dy)


# device time: 14734 ns/iter; 1.6916x vs baseline; 1.2491x over previous
import jax
import jax.numpy as jnp
from jax import lax
from jax.experimental import pallas as pl
from jax.experimental.pallas import tpu as pltpu

TM = 256
XS = 2
ZS = 4

_OFFSETS = tuple(
    (dx, dz) for dx in range(XS) for dz in range(ZS) if (dx, dz) != (0, 0)
)


def kernel(x, dy, gamma):
    del gamma
    m, d = x.shape
    m_loc = m // XS
    n_tiles = m_loc // TM
    n_peers = len(_OFFSETS)

    def body(x_hbm, dy_hbm, out_ref, xb, dyb, accum, recvs, ssems, rsems, csems):
        my_x = lax.axis_index("x")
        my_y = lax.axis_index("y")
        my_z = lax.axis_index("z")

        barrier = pltpu.get_barrier_semaphore()
        for dx, dz in _OFFSETS:
            pl.semaphore_signal(
                barrier,
                inc=1,
                device_id=((my_x + dx) % XS, my_y, (my_z + dz) % ZS),
                device_id_type=pl.DeviceIdType.MESH,
            )

        row0 = my_x * m_loc
        copies = []
        for t in range(n_tiles):
            cx = pltpu.make_async_copy(
                x_hbm.at[pl.ds(row0 + t * TM, TM), :], xb.at[t], csems.at[2 * t]
            )
            cd = pltpu.make_async_copy(
                dy_hbm.at[pl.ds(row0 + t * TM, TM), :],
                dyb.at[t],
                csems.at[2 * t + 1],
            )
            cx.start()
            cd.start()
            copies.append((cx, cd))

        total = jnp.zeros((2, d), jnp.float32)
        for t in range(n_tiles):
            cx, cd = copies[t]
            cx.wait()
            cd.wait()
            xt = xb[t]
            dyt = dyb[t]
            mu = jnp.mean(xt, axis=1, keepdims=True)
            xc = xt - mu
            var = jnp.mean(xc * xc, axis=1, keepdims=True)
            rstd = lax.rsqrt(var + 1e-5)
            xhat = xc * rstd
            dg = jnp.sum(dyt * xhat, axis=0, keepdims=True)
            db = jnp.sum(dyt, axis=0, keepdims=True)
            total = total + jnp.concatenate([dg, db], axis=0)
        accum[...] = total

        pl.semaphore_wait(barrier, n_peers)
        rdmas = []
        for s, (dx, dz) in enumerate(_OFFSETS):
            rdma = pltpu.make_async_remote_copy(
                src_ref=accum,
                dst_ref=recvs.at[s],
                send_sem=ssems.at[s],
                recv_sem=rsems.at[s],
                device_id=((my_x + dx) % XS, my_y, (my_z + dz) % ZS),
                device_id_type=pl.DeviceIdType.MESH,
            )
            rdma.start()
            rdmas.append(rdma)
        for s, rdma in enumerate(rdmas):
            rdma.wait_send()
            rdma.wait_recv()
            total = total + recvs[s]
        out_ref[...] = total

    return pl.pallas_call(
        body,
        in_specs=[
            pl.BlockSpec(memory_space=pltpu.MemorySpace.HBM),
            pl.BlockSpec(memory_space=pltpu.MemorySpace.HBM),
        ],
        out_specs=pl.BlockSpec(memory_space=pltpu.MemorySpace.VMEM),
        out_shape=jax.ShapeDtypeStruct((2, d), jnp.float32),
        scratch_shapes=[
            pltpu.VMEM((n_tiles, TM, d), jnp.float32),
            pltpu.VMEM((n_tiles, TM, d), jnp.float32),
            pltpu.VMEM((2, d), jnp.float32),
            pltpu.VMEM((n_peers, 2, d), jnp.float32),
            pltpu.SemaphoreType.DMA((n_peers,)),
            pltpu.SemaphoreType.DMA((n_peers,)),
            pltpu.SemaphoreType.DMA((2 * n_tiles,)),
        ],
        compiler_params=pltpu.CompilerParams(collective_id=0),
    )(
        pltpu.with_memory_space_constraint(x, pltpu.MemorySpace.HBM),
        pltpu.with_memory_space_constraint(dy, pltpu.MemorySpace.HBM),
    )
